# baseline (speedup 1.0000x reference)
import functools

import jax
import jax.numpy as jnp
from jax import lax
from jax.experimental import pallas as pl
from jax.experimental.pallas import tpu as pltpu
from jax.experimental.pallas import tpu_sc as plsc

NC, NS, L = 2, 16, 16
NW = NC * NS
N = 2048
B = 8
HALF = N // 2
NBLK = N // L
K = 16


def _round_bf16(v):
    u = plsc.bitcast(v, jnp.uint32)
    r = (u + jnp.uint32(0x7FFF) + ((u >> jnp.uint32(16)) & jnp.uint32(1)))
    r = r & jnp.uint32(0xFFFF0000)
    return plsc.bitcast(r, jnp.float32)


BIG = 3.0e38


def _knn_body(pts_hbm, out_hbm, cand_v, candr_v, cc_v, drow_v,
              buf0_v, buf1_v, buf2_v, buf3_v, acc_v):
    wid = lax.axis_index("s") * NC + lax.axis_index("c")
    prob = wid // 2
    half = wid % 2
    pltpu.sync_copy(pts_hbm.at[prob], cand_v)

    big_v = jnp.full((L,), BIG, dtype=jnp.float32)
    zero_v = jnp.zeros((L,), dtype=jnp.float32)

    def pre_body(j, carry):
        base = j * L
        cx = cand_v[0, pl.ds(base, L)]
        cy = cand_v[1, pl.ds(base, L)]
        cz = cand_v[2, pl.ds(base, L)]
        candr_v[0, pl.ds(base, L)] = _round_bf16(cx) * 2.0
        candr_v[1, pl.ds(base, L)] = _round_bf16(cy) * 2.0
        candr_v[2, pl.ds(base, L)] = _round_bf16(cz) * 2.0
        cc_v[pl.ds(base, L)] = (cx * cx + cy * cy) + cz * cz
        return carry

    lax.fori_loop(0, NBLK, pre_body, 0)

    def qblk_body(qb, acc_outer):
        qbase = half * HALF + qb * L
        qxb = cand_v[0, pl.ds(qbase, L)]
        qyb = cand_v[1, pl.ds(qbase, L)]
        qzb = cand_v[2, pl.ds(qbase, L)]
        qqb = (qxb * qxb + qyb * qyb) + qzb * qzb
        qxrb = _round_bf16(qxb)
        qyrb = _round_bf16(qyb)
        qzrb = _round_bf16(qzb)

        def splat(vec, lane):
            return jnp.full((L,), vec[lane], dtype=jnp.float32)

        def thr_of(w):
            ws, _ = plsc.sort_key_val(w, w)
            return jnp.full((L,), ws[L - 1], dtype=jnp.float32)

        bufs = [buf0_v, buf1_v, buf2_v, buf3_v]

        def pass_c(buf, cnt):
            buf[pl.ds(cnt, L)] = big_v
            nmerge = lax.div(cnt + (L - 1), jnp.int32(L))

            def body(t, top):
                blk = buf[pl.ds(t * L, L)]
                dcl = jnp.maximum(blk, zero_v)
                c_desc, _ = plsc.sort_key_val(dcl, dcl, descending=True)
                lo = jnp.minimum(top, c_desc)
                top_n, _ = plsc.sort_key_val(lo, lo)
                return top_n

            return lax.fori_loop(0, nmerge, body, big_v)

        R = 4
        acc = acc_outer
        for quad in range(L // R):
            ls = [R * quad + r for r in range(R)]
            qq = [splat(qqb, l) for l in ls]
            qx = [splat(qxrb, l) for l in ls]
            qy = [splat(qyrb, l) for l in ls]
            qz = [splat(qzrb, l) for l in ls]

            def pass_a(j, carry):
                ws = list(carry)
                for u in range(2):
                    base = (2 * j + u) * L
                    c2x = candr_v[0, pl.ds(base, L)]
                    c2y = candr_v[1, pl.ds(base, L)]
                    c2z = candr_v[2, pl.ds(base, L)]
                    cc = cc_v[pl.ds(base, L)]
                    for r in range(R):
                        i = (c2x * qx[r] + c2y * qy[r]) + c2z * qz[r]
                        d = (qq[r] - i) + cc
                        drow_v[r, pl.ds(base, L)] = d
                        ws[r] = jnp.minimum(ws[r], d)
                return tuple(ws)

            ws = lax.fori_loop(0, NBLK // 2, pass_a, (big_v,) * R)
            thrs = [thr_of(w) for w in ws]

            def pass_b(j, carry):
                cs = list(carry)
                for u in range(2):
                    base = (2 * j + u) * L
                    for r in range(R):
                        d = drow_v[r, pl.ds(base, L)]
                        m = d <= thrs[r]
                        plsc.store_compressed(
                            bufs[r].at[pl.ds(cs[r], L)], d, mask=m)
                        cs[r] = cs[r] + plsc.all_reduce_population_count(m)[0]
                return tuple(cs)

            cs = lax.fori_loop(0, NBLK // 2, pass_b, (jnp.int32(0),) * R)
            for r in range(R):
                acc = acc + pass_c(bufs[r], cs[r])
        return acc

    acc = lax.fori_loop(0, HALF // L, qblk_body,
                        jnp.zeros((L,), dtype=jnp.float32))
    acc_v[...] = acc
    pltpu.sync_copy(acc_v, out_hbm.at[wid])


_knn = functools.partial(
    pl.kernel,
    out_type=jax.ShapeDtypeStruct((NW, L), jnp.float32),
    mesh=plsc.VectorSubcoreMesh(core_axis_name="c", subcore_axis_name="s",
                                num_cores=NC, num_subcores=NS),
    scratch_types=[
        pltpu.VMEM((3, N), jnp.float32),
        pltpu.VMEM((3, N), jnp.float32),
        pltpu.VMEM((N,), jnp.float32),
        pltpu.VMEM((4, N), jnp.float32),
        pltpu.VMEM((N + L,), jnp.float32),
        pltpu.VMEM((N + L,), jnp.float32),
        pltpu.VMEM((N + L,), jnp.float32),
        pltpu.VMEM((N + L,), jnp.float32),
        pltpu.VMEM((L,), jnp.float32),
    ],
    compiler_params=pltpu.CompilerParams(needs_layout_passes=False),
)(_knn_body)


def kernel(seed, gt_s):
    pts = jnp.stack([seed, gt_s])
    pts = pts.transpose(0, 1, 3, 2).reshape(2 * B, 3, N)
    out = _knn(pts)
    per_prob = out.sum(axis=1).reshape(2 * B, 2).sum(axis=1)
    means = (per_prob / (N * K)).reshape(2, B)
    return jnp.mean((means[0] - means[1]) ** 2)


# speedup vs baseline: 1.0783x; 1.0783x over previous
import functools

import jax
import jax.numpy as jnp
from jax import lax
from jax.experimental import pallas as pl
from jax.experimental.pallas import tpu as pltpu
from jax.experimental.pallas import tpu_sc as plsc

NC, NS, L = 2, 16, 16
NW = NC * NS
N = 2048
B = 8
HALF = N // 2
NBLK = N // L
K = 16


def _round_bf16(v):
    u = plsc.bitcast(v, jnp.uint32)
    r = (u + jnp.uint32(0x7FFF) + ((u >> jnp.uint32(16)) & jnp.uint32(1)))
    r = r & jnp.uint32(0xFFFF0000)
    return plsc.bitcast(r, jnp.float32)


BIG = 3.0e38


def _knn_body(pts_hbm, out_hbm, cand_v, candr_v, cc_v, drow_v,
              buf0_v, buf1_v, buf2_v, buf3_v,
              buf4_v, buf5_v, buf6_v, buf7_v, acc_v):
    wid = lax.axis_index("s") * NC + lax.axis_index("c")
    prob = wid // 2
    half = wid % 2
    pltpu.sync_copy(pts_hbm.at[prob], cand_v)

    big_v = jnp.full((L,), BIG, dtype=jnp.float32)
    zero_v = jnp.zeros((L,), dtype=jnp.float32)

    def pre_body(j, carry):
        base = j * L
        cx = cand_v[0, pl.ds(base, L)]
        cy = cand_v[1, pl.ds(base, L)]
        cz = cand_v[2, pl.ds(base, L)]
        candr_v[0, pl.ds(base, L)] = _round_bf16(cx) * 2.0
        candr_v[1, pl.ds(base, L)] = _round_bf16(cy) * 2.0
        candr_v[2, pl.ds(base, L)] = _round_bf16(cz) * 2.0
        cc_v[pl.ds(base, L)] = (cx * cx + cy * cy) + cz * cz
        return carry

    lax.fori_loop(0, NBLK, pre_body, 0)

    def qblk_body(qb, acc_outer):
        qbase = half * HALF + qb * L
        qxb = cand_v[0, pl.ds(qbase, L)]
        qyb = cand_v[1, pl.ds(qbase, L)]
        qzb = cand_v[2, pl.ds(qbase, L)]
        qqb = (qxb * qxb + qyb * qyb) + qzb * qzb
        qxrb = _round_bf16(qxb)
        qyrb = _round_bf16(qyb)
        qzrb = _round_bf16(qzb)

        def splat(vec, lane):
            return jnp.full((L,), vec[lane], dtype=jnp.float32)

        def thr_of(w):
            ws, _ = plsc.sort_key_val(w, w)
            return jnp.full((L,), ws[L - 1], dtype=jnp.float32)

        bufs = [buf0_v, buf1_v, buf2_v, buf3_v,
                buf4_v, buf5_v, buf6_v, buf7_v]

        def pass_c(buf, cnt):
            buf[pl.ds(cnt, L)] = big_v
            nmerge = lax.div(cnt + (L - 1), jnp.int32(L))

            def body(t, top):
                blk = buf[pl.ds(t * L, L)]
                dcl = jnp.maximum(blk, zero_v)
                c_desc, _ = plsc.sort_key_val(dcl, dcl, descending=True)
                lo = jnp.minimum(top, c_desc)
                top_n, _ = plsc.sort_key_val(lo, lo)
                return top_n

            return lax.fori_loop(0, nmerge, body, big_v)

        R = 8
        acc = acc_outer
        for quad in range(L // R):
            ls = [R * quad + r for r in range(R)]
            qq = [splat(qqb, l) for l in ls]
            qx = [splat(qxrb, l) for l in ls]
            qy = [splat(qyrb, l) for l in ls]
            qz = [splat(qzrb, l) for l in ls]

            def pass_a(j, carry):
                ws = list(carry)
                for u in range(2):
                    base = (2 * j + u) * L
                    c2x = candr_v[0, pl.ds(base, L)]
                    c2y = candr_v[1, pl.ds(base, L)]
                    c2z = candr_v[2, pl.ds(base, L)]
                    cc = cc_v[pl.ds(base, L)]
                    for r in range(R):
                        i = (c2x * qx[r] + c2y * qy[r]) + c2z * qz[r]
                        d = (qq[r] - i) + cc
                        drow_v[r, pl.ds(base, L)] = d
                        ws[r] = jnp.minimum(ws[r], d)
                return tuple(ws)

            ws = lax.fori_loop(0, NBLK // 2, pass_a, (big_v,) * R)
            thrs = [thr_of(w) for w in ws]

            def pass_b(j, carry):
                cs = list(carry)
                for u in range(2):
                    base = (2 * j + u) * L
                    for r in range(R):
                        d = drow_v[r, pl.ds(base, L)]
                        m = d <= thrs[r]
                        plsc.store_compressed(
                            bufs[r].at[pl.ds(cs[r], L)], d, mask=m)
                        cs[r] = cs[r] + plsc.all_reduce_population_count(m)[0]
                return tuple(cs)

            cs = lax.fori_loop(0, NBLK // 2, pass_b, (jnp.int32(0),) * R)
            for r in range(R):
                acc = acc + pass_c(bufs[r], cs[r])
        return acc

    acc = lax.fori_loop(0, HALF // L, qblk_body,
                        jnp.zeros((L,), dtype=jnp.float32))
    acc_v[...] = acc
    pltpu.sync_copy(acc_v, out_hbm.at[wid])


_knn = functools.partial(
    pl.kernel,
    out_type=jax.ShapeDtypeStruct((NW, L), jnp.float32),
    mesh=plsc.VectorSubcoreMesh(core_axis_name="c", subcore_axis_name="s",
                                num_cores=NC, num_subcores=NS),
    scratch_types=[
        pltpu.VMEM((3, N), jnp.float32),
        pltpu.VMEM((3, N), jnp.float32),
        pltpu.VMEM((N,), jnp.float32),
        pltpu.VMEM((8, N), jnp.float32),
        pltpu.VMEM((N + L,), jnp.float32),
        pltpu.VMEM((N + L,), jnp.float32),
        pltpu.VMEM((N + L,), jnp.float32),
        pltpu.VMEM((N + L,), jnp.float32),
        pltpu.VMEM((N + L,), jnp.float32),
        pltpu.VMEM((N + L,), jnp.float32),
        pltpu.VMEM((N + L,), jnp.float32),
        pltpu.VMEM((N + L,), jnp.float32),
        pltpu.VMEM((L,), jnp.float32),
    ],
    compiler_params=pltpu.CompilerParams(needs_layout_passes=False,
                                        disable_bounds_checks=True),
)(_knn_body)


def kernel(seed, gt_s):
    pts = jnp.stack([seed, gt_s])
    pts = pts.transpose(0, 1, 3, 2).reshape(2 * B, 3, N)
    out = _knn(pts)
    per_prob = out.sum(axis=1).reshape(2 * B, 2).sum(axis=1)
    means = (per_prob / (N * K)).reshape(2, B)
    return jnp.mean((means[0] - means[1]) ** 2)
